# baseline (speedup 1.0000x reference)
import jax
import jax.numpy as jnp
from jax import lax
from jax.experimental import pallas as pl
from jax.experimental.pallas import tpu as pltpu
from jax.experimental.pallas import tpu_sc as plsc

BATCH = 16384
EMBED = 64
NUM_CORES = 2
NUM_SUBCORES = 16
NUM_WORKERS = NUM_CORES * NUM_SUBCORES
BPW = BATCH // NUM_WORKERS
NCHUNK = 4
CH = BPW // NCHUNK
LANES = 16
GPC = CH // LANES


def _body(uidx_hbm, midx_hbm, uemb_hbm, ubg_hbm, memb_hbm, mbg_hbm,
          out_hbm, uidx_v, midx_v, urows_v, mrows_v, ub_v, mb_v, res_v,
          sem_rows):
    c = lax.axis_index("c")
    s = lax.axis_index("s")
    wid = s * NUM_CORES + c
    base = wid * BPW

    for j in range(NCHUNK):
        pltpu.sync_copy(uidx_hbm.at[pl.ds(base + j * CH, CH)], uidx_v.at[j])
        pltpu.sync_copy(midx_hbm.at[pl.ds(base + j * CH, CH)], midx_v.at[j])
        pltpu.sync_copy(ubg_hbm.at[pl.ds(base + j * CH, CH)], ub_v.at[j])
        pltpu.sync_copy(mbg_hbm.at[pl.ds(base + j * CH, CH)], mb_v.at[j])

    lane = lax.iota(jnp.int32, LANES)
    perms = [lane ^ sh for sh in (8, 4, 2, 1)]

    for j in range(NCHUNK):
        def fire_body(g, carry, j=j):
            goff = pl.multiple_of(g * LANES, LANES)
            uv = uidx_v[j, pl.ds(goff, LANES)]
            mv = midx_v[j, pl.ds(goff, LANES)]
            for i in range(LANES):
                r = goff + i
                pltpu.async_copy(uemb_hbm.at[pl.ds(uv[i], 1)],
                                 urows_v.at[pl.ds(r, 1)], sem_rows)
                pltpu.async_copy(memb_hbm.at[pl.ds(mv[i], 1)],
                                 mrows_v.at[pl.ds(r, 1)], sem_rows)
            return carry
        lax.fori_loop(0, GPC, fire_body, 0)

        pltpu.make_async_copy(uemb_hbm.at[pl.ds(0, CH)], urows_v, sem_rows).wait()
        pltpu.make_async_copy(memb_hbm.at[pl.ds(0, CH)], mrows_v, sem_rows).wait()

        def group_body(g, carry, j=j):
            goff = pl.multiple_of(g * LANES, LANES)
            vec = jnp.zeros((LANES,), jnp.float32)
            for i in range(LANES):
                r = goff + i
                acc = None
                for k in range(EMBED // LANES):
                    u = urows_v[r, pl.ds(k * LANES, LANES)]
                    m = mrows_v[r, pl.ds(k * LANES, LANES)]
                    p = u * m
                    acc = p if acc is None else acc + p
                for perm in perms:
                    acc = acc + acc.at[perm].get(mode="promise_in_bounds")
                vec = jnp.where(lane == i, acc, vec)
            x = vec + ub_v[j, pl.ds(goff, LANES)] + mb_v[j, pl.ds(goff, LANES)]
            y = 1.0 / (1.0 + jnp.exp(-x))
            res_v[pl.ds(j * CH + goff, LANES)] = y
            return carry
        lax.fori_loop(0, GPC, group_body, 0)

    pltpu.sync_copy(res_v, out_hbm.at[pl.ds(base, BPW)])


@jax.jit
def _run(uidx, midx, uemb, ubias, memb, mbias):
    mesh = plsc.VectorSubcoreMesh(core_axis_name="c", subcore_axis_name="s")
    kfn = pl.kernel(
        _body,
        mesh=mesh,
        compiler_params=pltpu.CompilerParams(use_tc_tiling_on_sc=True,
                                             needs_layout_passes=False),
        out_type=jax.ShapeDtypeStruct((BATCH,), jnp.float32),
        scratch_types=[
            pltpu.VMEM((NCHUNK, CH), jnp.int32),
            pltpu.VMEM((NCHUNK, CH), jnp.int32),
            pltpu.VMEM((CH, EMBED), jnp.float32),
            pltpu.VMEM((CH, EMBED), jnp.float32),
            pltpu.VMEM((NCHUNK, CH), jnp.float32),
            pltpu.VMEM((NCHUNK, CH), jnp.float32),
            pltpu.VMEM((BPW,), jnp.float32),
            pltpu.SemaphoreType.DMA,
        ],
    )
    return kfn(uidx, midx, uemb, ubias, memb, mbias)


def kernel(user_input, movie_input, user_embedding, user_bias,
           movie_embedding, movie_bias):
    ubg2 = jnp.take(user_bias, user_input, axis=0)
    mbg2 = jnp.take(movie_bias, movie_input, axis=0)
    ubg2, mbg2 = lax.optimization_barrier((ubg2, mbg2))
    ubg = ubg2.reshape(-1)
    mbg = mbg2.reshape(-1)
    return _run(user_input.astype(jnp.int32), movie_input.astype(jnp.int32),
                user_embedding, ubg, movie_embedding, mbg)


# speedup vs baseline: 1.0278x; 1.0278x over previous
import jax
import jax.numpy as jnp
from jax import lax
from jax.experimental import pallas as pl
from jax.experimental.pallas import tpu as pltpu
from jax.experimental.pallas import tpu_sc as plsc

BATCH = 16384
EMBED = 64
NUM_CORES = 2
NUM_SUBCORES = 16
NUM_WORKERS = NUM_CORES * NUM_SUBCORES
BPW = BATCH // NUM_WORKERS
NCHUNK = 4
CH = BPW // NCHUNK
LANES = 16
GPC = CH // LANES
BROWS = CH // EMBED


def _body(uidx_hbm, midx_hbm, uemb_hbm, memb_hbm,
          out_hbm, uidx_v, midx_v, urows_v, mrows_v, res_v, sem_rows):
    c = lax.axis_index("c")
    s = lax.axis_index("s")
    wid = s * NUM_CORES + c
    base = wid * BPW

    for j in range(NCHUNK):
        pltpu.sync_copy(uidx_hbm.at[pl.ds(base + j * CH, CH)], uidx_v.at[j])
        pltpu.sync_copy(midx_hbm.at[pl.ds(base + j * CH, CH)], midx_v.at[j])

    lane = lax.iota(jnp.int32, LANES)
    perms = [lane ^ sh for sh in (8, 4, 2, 1)]

    for j in range(NCHUNK):
        def fire_body(g, carry, j=j):
            goff = pl.multiple_of(g * LANES, LANES)
            uv = uidx_v[j, pl.ds(goff, LANES)]
            mv = midx_v[j, pl.ds(goff, LANES)]
            for i in range(LANES):
                r = goff + i
                pltpu.async_copy(uemb_hbm.at[pl.ds(uv[i], 1)],
                                 urows_v.at[pl.ds(r, 1)], sem_rows)
                pltpu.async_copy(memb_hbm.at[pl.ds(mv[i], 1)],
                                 mrows_v.at[pl.ds(r, 1)], sem_rows)
            return carry
        lax.fori_loop(0, GPC, fire_body, 0)

        pltpu.make_async_copy(uemb_hbm.at[pl.ds(0, CH)], urows_v, sem_rows).wait()
        pltpu.make_async_copy(memb_hbm.at[pl.ds(0, CH)], mrows_v, sem_rows).wait()

        def group_body(g, carry, j=j):
            goff = pl.multiple_of(g * LANES, LANES)
            vec = jnp.zeros((LANES,), jnp.float32)
            for i in range(LANES):
                r = goff + i
                acc = None
                for k in range(EMBED // LANES):
                    u = urows_v[r, pl.ds(k * LANES, LANES)]
                    m = mrows_v[r, pl.ds(k * LANES, LANES)]
                    p = u * m
                    acc = p if acc is None else acc + p
                for perm in perms:
                    acc = acc + acc.at[perm].get(mode="promise_in_bounds")
                vec = jnp.where(lane == i, acc, vec)
            res_v[pl.ds(j * CH + goff, LANES)] = vec
            return carry
        lax.fori_loop(0, GPC, group_body, 0)

    pltpu.sync_copy(res_v, out_hbm.at[pl.ds(base, BPW)])


@jax.jit
def _run(uidx, midx, uemb, memb):
    mesh = plsc.VectorSubcoreMesh(core_axis_name="c", subcore_axis_name="s")
    kfn = pl.kernel(
        _body,
        mesh=mesh,
        compiler_params=pltpu.CompilerParams(use_tc_tiling_on_sc=True,
                                             needs_layout_passes=False),
        out_type=jax.ShapeDtypeStruct((BATCH,), jnp.float32),
        scratch_types=[
            pltpu.VMEM((NCHUNK, CH), jnp.int32),
            pltpu.VMEM((NCHUNK, CH), jnp.int32),
            pltpu.VMEM((CH, EMBED), jnp.float32),
            pltpu.VMEM((CH, EMBED), jnp.float32),
            pltpu.VMEM((BPW,), jnp.float32),
            pltpu.SemaphoreType.DMA,
        ],
    )
    return kfn(uidx, midx, uemb, memb)


def kernel(user_input, movie_input, user_embedding, user_bias,
           movie_embedding, movie_bias):
    dots = _run(user_input.astype(jnp.int32), movie_input.astype(jnp.int32),
                user_embedding, movie_embedding)
    ub = jnp.take(user_bias, user_input, axis=0)
    mb = jnp.take(movie_bias, movie_input, axis=0)
    return jax.nn.sigmoid(dots + jnp.squeeze(ub, axis=-1)
                          + jnp.squeeze(mb, axis=-1))


# speedup vs baseline: 1.0937x; 1.0642x over previous
import jax
import jax.numpy as jnp
from jax import lax
from jax.experimental import pallas as pl
from jax.experimental.pallas import tpu as pltpu
from jax.experimental.pallas import tpu_sc as plsc

BATCH = 16384
EMBED = 64
NUM_CORES = 2
NUM_SUBCORES = 16
NUM_WORKERS = NUM_CORES * NUM_SUBCORES
BPW = BATCH // NUM_WORKERS
NCHUNK = 4
CH = BPW // NCHUNK
LANES = 16
GPC = CH // LANES


def _body(uidx_hbm, midx_hbm, uemb_hbm, ubg_hbm, memb_hbm, mbg_hbm,
          out_hbm, uidx_v, midx_v, urows_v, mrows_v, ub_v, mb_v, res_v,
          sem_rows):
    c = lax.axis_index("c")
    s = lax.axis_index("s")
    wid = s * NUM_CORES + c
    base = wid * BPW

    for j in range(NCHUNK):
        pltpu.sync_copy(uidx_hbm.at[pl.ds(base + j * CH, CH)], uidx_v.at[j])
        pltpu.sync_copy(midx_hbm.at[pl.ds(base + j * CH, CH)], midx_v.at[j])
        pltpu.sync_copy(ubg_hbm.at[pl.ds(base + j * CH, CH)], ub_v.at[j])
        pltpu.sync_copy(mbg_hbm.at[pl.ds(base + j * CH, CH)], mb_v.at[j])

    lane = lax.iota(jnp.int32, LANES)
    perms = [lane ^ sh for sh in (8, 4, 2, 1)]

    for j in range(NCHUNK):
        def fire_body(g, carry, j=j):
            goff = pl.multiple_of(g * LANES, LANES)
            uv = uidx_v[j, pl.ds(goff, LANES)]
            mv = midx_v[j, pl.ds(goff, LANES)]
            for i in range(LANES):
                r = goff + i
                pltpu.async_copy(uemb_hbm.at[pl.ds(uv[i], 1)],
                                 urows_v.at[pl.ds(r, 1)], sem_rows)
                pltpu.async_copy(memb_hbm.at[pl.ds(mv[i], 1)],
                                 mrows_v.at[pl.ds(r, 1)], sem_rows)
            return carry
        lax.fori_loop(0, GPC, fire_body, 0)

        pltpu.make_async_copy(uemb_hbm.at[pl.ds(0, CH)], urows_v, sem_rows).wait()
        pltpu.make_async_copy(memb_hbm.at[pl.ds(0, CH)], mrows_v, sem_rows).wait()

        def group_body(g, carry, j=j):
            goff = pl.multiple_of(g * LANES, LANES)
            vec = jnp.zeros((LANES,), jnp.float32)
            for i in range(LANES):
                r = goff + i
                acc = None
                for k in range(EMBED // LANES):
                    u = urows_v[r, pl.ds(k * LANES, LANES)]
                    m = mrows_v[r, pl.ds(k * LANES, LANES)]
                    p = u * m
                    acc = p if acc is None else acc + p
                for perm in perms:
                    acc = acc + acc.at[perm].get(mode="promise_in_bounds")
                vec = jnp.where(lane == i, acc, vec)
            x = vec + ub_v[j, pl.ds(goff, LANES)] + mb_v[j, pl.ds(goff, LANES)]
            y = 1.0 / (1.0 + jnp.exp(-x))
            res_v[pl.ds(j * CH + goff, LANES)] = y
            return carry
        lax.fori_loop(0, GPC, group_body, 0)

    pltpu.sync_copy(res_v, out_hbm.at[pl.ds(base, BPW)])


@jax.jit
def _run(uidx, midx, uemb, ubg, memb, mbg):
    mesh = plsc.VectorSubcoreMesh(core_axis_name="c", subcore_axis_name="s")
    kfn = pl.kernel(
        _body,
        mesh=mesh,
        compiler_params=pltpu.CompilerParams(use_tc_tiling_on_sc=True,
                                             needs_layout_passes=False),
        out_type=jax.ShapeDtypeStruct((BATCH,), jnp.float32),
        scratch_types=[
            pltpu.VMEM((NCHUNK, CH), jnp.int32),
            pltpu.VMEM((NCHUNK, CH), jnp.int32),
            pltpu.VMEM((CH, EMBED), jnp.float32),
            pltpu.VMEM((CH, EMBED), jnp.float32),
            pltpu.VMEM((NCHUNK, CH), jnp.float32),
            pltpu.VMEM((NCHUNK, CH), jnp.float32),
            pltpu.VMEM((BPW,), jnp.float32),
            pltpu.SemaphoreType.DMA,
        ],
    )
    return kfn(uidx, midx, uemb, ubg, memb, mbg)


def kernel(user_input, movie_input, user_embedding, user_bias,
           movie_embedding, movie_bias):
    ubg = user_bias[user_input, 0]
    mbg = movie_bias[movie_input, 0]
    return _run(user_input.astype(jnp.int32), movie_input.astype(jnp.int32),
                user_embedding, ubg, movie_embedding, mbg)
